# Initial kernel scaffold; baseline (speedup 1.0000x reference)
import jax
import jax.numpy as jnp
from jax.experimental import pallas as pl


def kernel(dense_inputs, sparse_inputs, tables, W_lin, b_lin, W1, b1, W2, b2, W3, b3, W4, b4):
    raise NotImplementedError("write your pallas kernel here")


# speedup vs baseline: 7.3102x; 7.3102x over previous
import functools

import jax
import jax.numpy as jnp
from jax.experimental import pallas as pl
from jax.experimental.pallas import tpu as pltpu
from jax.experimental.pallas import tpu_sc as plsc

B = 16384
F = 26
V = 100000
D = 16
EO = F * D
F_DENSE = 13

_GATHER_WINDOW = 128


def _sc_gather(flat_tables, flat_idx):
    n = B * F
    mesh = plsc.VectorSubcoreMesh(core_axis_name="core", subcore_axis_name="subcore")

    @functools.partial(
        pl.kernel,
        out_type=jax.ShapeDtypeStruct((n, D), flat_tables.dtype),
        mesh=mesh,
        compiler_params=pltpu.CompilerParams(use_tc_tiling_on_sc=False),
    )
    def gather_kernel(x_hbm, i_hbm, o_hbm):
        def body(i_vmem, o_vmem):
            pltpu.sync_copy(x_hbm.at[i_vmem.at[0]], o_vmem)

        pltpu.emit_pipeline(
            body,
            grid=(n // _GATHER_WINDOW,),
            in_specs=[pl.BlockSpec((1, _GATHER_WINDOW), index_map=lambda i: (0, i))],
            out_specs=[pl.BlockSpec((_GATHER_WINDOW, D), index_map=lambda i: (i, 0))],
            core_axis_name=("core", "subcore"),
            dimension_semantics=(pltpu.PARALLEL,),
        )(i_hbm, o_hbm)

    return gather_kernel(flat_tables, flat_idx)


def _mlp_body(g_ref, d_ref, wlint_ref, w1e_ref, w1d_ref, b1_ref, w2_ref,
              b2_ref, w3_ref, b3_ref, w4t_ref, bias_ref, o_ref):
    g32 = g_ref[...]
    gb = g32.astype(jnp.bfloat16)
    db = d_ref[...].astype(jnp.bfloat16)
    wide = jnp.sum(g32 * wlint_ref[...], axis=1, keepdims=True)
    h = jnp.dot(gb, w1e_ref[...], preferred_element_type=jnp.float32)
    h = h + jnp.dot(db, w1d_ref[...], preferred_element_type=jnp.float32)
    h = jnp.maximum(h + b1_ref[...], 0.0)
    h = jnp.dot(h.astype(jnp.bfloat16), w2_ref[...], preferred_element_type=jnp.float32)
    h = jnp.maximum(h + b2_ref[...], 0.0)
    h = jnp.dot(h.astype(jnp.bfloat16), w3_ref[...], preferred_element_type=jnp.float32)
    h = jnp.maximum(h + b3_ref[...], 0.0)
    deep = jnp.sum(h * w4t_ref[...], axis=1, keepdims=True)
    o_ref[...] = jax.nn.sigmoid(0.5 * (wide + deep + bias_ref[...]))


def _mlp_tc(gathered, dense_inputs, wlint, w1e, w1d, b1, w2, b2, w3, b3, w4t, bias):
    bt = 2048
    grid = (B // bt,)

    def _const(shape):
        nd = len(shape)
        return pl.BlockSpec(shape, lambda i, _nd=nd: (0,) * _nd)

    return pl.pallas_call(
        _mlp_body,
        grid=grid,
        in_specs=[
            pl.BlockSpec((bt, EO), lambda i: (i, 0)),
            pl.BlockSpec((bt, F_DENSE), lambda i: (i, 0)),
            _const((1, EO)),
            _const((EO, 1024)),
            _const((F_DENSE, 1024)),
            _const((1, 1024)),
            _const((1024, 512)),
            _const((1, 512)),
            _const((512, 256)),
            _const((1, 256)),
            _const((1, 256)),
            _const((1, 1)),
        ],
        out_specs=pl.BlockSpec((bt, 1), lambda i: (i, 0)),
        out_shape=jax.ShapeDtypeStruct((B, 1), jnp.float32),
        compiler_params=pltpu.CompilerParams(
            dimension_semantics=("arbitrary",),
        ),
    )(gathered, dense_inputs, wlint, w1e, w1d, b1, w2, b2, w3, b3, w4t, bias)


def kernel(dense_inputs, sparse_inputs, tables, W_lin, b_lin, W1, b1, W2, b2, W3, b3, W4, b4):
    flat_tables = tables.reshape(F * V, D)
    flat_idx = (sparse_inputs + jnp.arange(F, dtype=jnp.int32)[None, :] * V).reshape(1, B * F)
    gathered = _sc_gather(flat_tables, flat_idx)
    gathered = gathered.reshape(B, EO)

    w1e = W1[:EO].astype(jnp.bfloat16)
    w1d = W1[EO:].astype(jnp.bfloat16)
    w2 = W2.astype(jnp.bfloat16)
    w3 = W3.astype(jnp.bfloat16)
    bias = (b_lin + b4).reshape(1, 1)
    return _mlp_tc(
        gathered, dense_inputs,
        W_lin.reshape(1, EO), w1e, w1d, b1.reshape(1, 1024),
        w2, b2.reshape(1, 512), w3, b3.reshape(1, 256),
        W4.reshape(1, 256), bias,
    )


# speedup vs baseline: 31.7851x; 4.3480x over previous
import functools

import jax
import jax.numpy as jnp
from jax import lax
from jax.experimental import pallas as pl
from jax.experimental.pallas import tpu as pltpu
from jax.experimental.pallas import tpu_sc as plsc

B = 16384
F = 26
V = 100000
D = 16
EO = F * D
F_DENSE = 13

_NW = 32
_RPW = EO // _NW
_OUT_CHUNK = 8192


def _sc_gather_t(tviewT, idxT):
    mesh = plsc.VectorSubcoreMesh(core_axis_name="core", subcore_axis_name="subcore")

    @functools.partial(
        pl.kernel,
        out_type=jax.ShapeDtypeStruct((EO, B), jnp.float32),
        mesh=mesh,
        scratch_types=[
            pltpu.VMEM((V,), jnp.float32),
            pltpu.VMEM((B,), jnp.int32),
            pltpu.VMEM((_OUT_CHUNK,), jnp.float32),
        ],
        compiler_params=pltpu.CompilerParams(needs_layout_passes=False),
    )
    def gather_kernel(t_hbm, i_hbm, o_hbm, row_v, idx_v, out_v):
        cid = lax.axis_index("core")
        sid = lax.axis_index("subcore")
        wid = sid * 2 + cid

        @pl.loop(0, _RPW)
        def _(k):
            r = wid * _RPW + k
            f = r // D
            pltpu.sync_copy(t_hbm.at[r], row_v)
            pltpu.sync_copy(i_hbm.at[f], idx_v)

            @pl.loop(0, B // _OUT_CHUNK)
            def _(c):
                @pl.loop(0, _OUT_CHUNK // 16, unroll=8)
                def _(j):
                    vidx = idx_v[pl.ds(c * _OUT_CHUNK + j * 16, 16)]
                    out_v[pl.ds(j * 16, 16)] = plsc.load_gather(row_v, [vidx])

                pltpu.sync_copy(out_v, o_hbm.at[r, pl.ds(c * _OUT_CHUNK, _OUT_CHUNK)])

    return gather_kernel(tviewT, idxT)


def _mlp_body_t(g_ref, d_ref, wlin_ref, w1et_ref, w1dt_ref, b1_ref, w2t_ref,
                b2_ref, w3t_ref, b3_ref, w4_ref, bias_ref, o_ref):
    g32 = g_ref[...]
    gb = g32.astype(jnp.bfloat16)
    db = d_ref[...].astype(jnp.bfloat16)
    wide = jnp.sum(g32 * wlin_ref[...], axis=0, keepdims=True)
    h = jnp.dot(w1et_ref[...], gb, preferred_element_type=jnp.float32)
    h = h + jnp.dot(w1dt_ref[...], db, preferred_element_type=jnp.float32)
    h = jnp.maximum(h + b1_ref[...], 0.0)
    h = jnp.dot(w2t_ref[...], h.astype(jnp.bfloat16), preferred_element_type=jnp.float32)
    h = jnp.maximum(h + b2_ref[...], 0.0)
    h = jnp.dot(w3t_ref[...], h.astype(jnp.bfloat16), preferred_element_type=jnp.float32)
    h = jnp.maximum(h + b3_ref[...], 0.0)
    deep = jnp.sum(h * w4_ref[...], axis=0, keepdims=True)
    o_ref[...] = jax.nn.sigmoid(0.5 * (wide + deep + bias_ref[...]))


def _mlp_tc_t(embT, dT, wlin, w1et, w1dt, b1c, w2t, b2c, w3t, b3c, w4, bias):
    bt = 2048
    grid = (B // bt,)

    def _const(shape):
        nd = len(shape)
        return pl.BlockSpec(shape, lambda i, _nd=nd: (0,) * _nd)

    return pl.pallas_call(
        _mlp_body_t,
        grid=grid,
        in_specs=[
            pl.BlockSpec((EO, bt), lambda i: (0, i)),
            pl.BlockSpec((F_DENSE, bt), lambda i: (0, i)),
            _const((EO, 1)),
            _const((1024, EO)),
            _const((1024, F_DENSE)),
            _const((1024, 1)),
            _const((512, 1024)),
            _const((512, 1)),
            _const((256, 512)),
            _const((256, 1)),
            _const((256, 1)),
            _const((1, 1)),
        ],
        out_specs=pl.BlockSpec((1, bt), lambda i: (0, i)),
        out_shape=jax.ShapeDtypeStruct((1, B), jnp.float32),
        compiler_params=pltpu.CompilerParams(
            dimension_semantics=("arbitrary",),
        ),
    )(embT, dT, wlin, w1et, w1dt, b1c, w2t, b2c, w3t, b3c, w4, bias)


def kernel(dense_inputs, sparse_inputs, tables, W_lin, b_lin, W1, b1, W2, b2, W3, b3, W4, b4):
    tviewT = jnp.transpose(tables, (0, 2, 1)).reshape(EO, V)
    idxT = jnp.transpose(sparse_inputs)
    dT = jnp.transpose(dense_inputs)

    embT = _sc_gather_t(tviewT, idxT)

    w1et = W1[:EO].T.astype(jnp.bfloat16)
    w1dt = W1[EO:].T.astype(jnp.bfloat16)
    w2t = W2.T.astype(jnp.bfloat16)
    w3t = W3.T.astype(jnp.bfloat16)
    bias = (b_lin + b4).reshape(1, 1)
    outT = _mlp_tc_t(
        embT, dT, W_lin, w1et, w1dt, b1.reshape(1024, 1),
        w2t, b2.reshape(512, 1), w3t, b3.reshape(256, 1),
        W4, bias,
    )
    return outT.reshape(B, 1)


# speedup vs baseline: 33.2807x; 1.0471x over previous
import functools

import jax
import jax.numpy as jnp
from jax import lax
from jax.experimental import pallas as pl
from jax.experimental.pallas import tpu as pltpu
from jax.experimental.pallas import tpu_sc as plsc

B = 16384
F = 26
V = 100000
D = 16
EO = F * D
F_DENSE = 13

_NW = 32
_RPW = EO // _NW
_OUT_CHUNK = 4096


def _sc_gather_t(tviewT, idxT):
    mesh = plsc.VectorSubcoreMesh(core_axis_name="core", subcore_axis_name="subcore")

    nchunk = B // _OUT_CHUNK

    @functools.partial(
        pl.kernel,
        out_type=jax.ShapeDtypeStruct((EO, B), jnp.float32),
        mesh=mesh,
        scratch_types=[
            pltpu.VMEM((V,), jnp.float32),
            pltpu.VMEM((B,), jnp.int32),
            pltpu.VMEM((_OUT_CHUNK,), jnp.float32),
            pltpu.VMEM((_OUT_CHUNK,), jnp.float32),
            pltpu.SemaphoreType.DMA,
            pltpu.SemaphoreType.DMA,
            pltpu.SemaphoreType.DMA,
            pltpu.SemaphoreType.DMA,
        ],
        compiler_params=pltpu.CompilerParams(needs_layout_passes=False),
    )
    def gather_kernel(t_hbm, i_hbm, o_hbm, row_v, idx_v, out0_v, out1_v,
                      sem_row, sem_idx, sem_o0, sem_o1):
        cid = lax.axis_index("core")
        sid = lax.axis_index("subcore")
        wid = sid * 2 + cid
        outs = (out0_v, out1_v)
        osems = (sem_o0, sem_o1)

        @pl.loop(0, _RPW)
        def _(k):
            r = wid * _RPW + k
            f = r // D
            rc = pltpu.async_copy(t_hbm.at[r], row_v, sem_row)
            ic = pltpu.async_copy(i_hbm.at[f], idx_v, sem_idx)
            ic.wait()
            rc.wait()

            for c in range(nchunk):
                ob = outs[c % 2]
                osem = osems[c % 2]
                if c >= 2:
                    pltpu.make_async_copy(
                        ob, o_hbm.at[r, pl.ds((c - 2) * _OUT_CHUNK, _OUT_CHUNK)], osem
                    ).wait()
                else:
                    @pl.when(k > 0)
                    def _():
                        pltpu.make_async_copy(
                            ob, o_hbm.at[r, pl.ds(c * _OUT_CHUNK, _OUT_CHUNK)], osem
                        ).wait()

                @pl.loop(0, _OUT_CHUNK // 16, unroll=8)
                def _(j):
                    vidx = idx_v[pl.ds(c * _OUT_CHUNK + j * 16, 16)]
                    ob[pl.ds(j * 16, 16)] = plsc.load_gather(row_v, [vidx])

                pltpu.async_copy(
                    ob, o_hbm.at[r, pl.ds(c * _OUT_CHUNK, _OUT_CHUNK)], osem)

        last_r = wid * _RPW + _RPW - 1
        for c in (nchunk - 2, nchunk - 1):
            pltpu.make_async_copy(
                outs[c % 2],
                o_hbm.at[last_r, pl.ds(c * _OUT_CHUNK, _OUT_CHUNK)],
                osems[c % 2],
            ).wait()

    return gather_kernel(tviewT, idxT)


def _mlp_body_t(g_ref, d_ref, wlin_ref, w1et_ref, w1dt_ref, b1_ref, w2t_ref,
                b2_ref, w3t_ref, b3_ref, w4_ref, bias_ref, o_ref):
    g32 = g_ref[...]
    gb = g32.astype(jnp.bfloat16)
    db = d_ref[...].astype(jnp.bfloat16)
    wide = jnp.sum(g32 * wlin_ref[...], axis=0, keepdims=True)
    h = jnp.dot(w1et_ref[...], gb, preferred_element_type=jnp.float32)
    h = h + jnp.dot(w1dt_ref[...], db, preferred_element_type=jnp.float32)
    h = jnp.maximum(h + b1_ref[...], 0.0)
    h = jnp.dot(w2t_ref[...], h.astype(jnp.bfloat16), preferred_element_type=jnp.float32)
    h = jnp.maximum(h + b2_ref[...], 0.0)
    h = jnp.dot(w3t_ref[...], h.astype(jnp.bfloat16), preferred_element_type=jnp.float32)
    h = jnp.maximum(h + b3_ref[...], 0.0)
    deep = jnp.sum(h * w4_ref[...], axis=0, keepdims=True)
    o_ref[...] = jax.nn.sigmoid(0.5 * (wide + deep + bias_ref[...]))


def _mlp_tc_t(embT, dT, wlin, w1et, w1dt, b1c, w2t, b2c, w3t, b3c, w4, bias):
    bt = 2048
    grid = (B // bt,)

    def _const(shape):
        nd = len(shape)
        return pl.BlockSpec(shape, lambda i, _nd=nd: (0,) * _nd)

    return pl.pallas_call(
        _mlp_body_t,
        grid=grid,
        in_specs=[
            pl.BlockSpec((EO, bt), lambda i: (0, i)),
            pl.BlockSpec((F_DENSE, bt), lambda i: (0, i)),
            _const((EO, 1)),
            _const((1024, EO)),
            _const((1024, F_DENSE)),
            _const((1024, 1)),
            _const((512, 1024)),
            _const((512, 1)),
            _const((256, 512)),
            _const((256, 1)),
            _const((256, 1)),
            _const((1, 1)),
        ],
        out_specs=pl.BlockSpec((1, bt), lambda i: (0, i)),
        out_shape=jax.ShapeDtypeStruct((1, B), jnp.float32),
        compiler_params=pltpu.CompilerParams(
            dimension_semantics=("arbitrary",),
        ),
    )(embT, dT, wlin, w1et, w1dt, b1c, w2t, b2c, w3t, b3c, w4, bias)


def kernel(dense_inputs, sparse_inputs, tables, W_lin, b_lin, W1, b1, W2, b2, W3, b3, W4, b4):
    tviewT = jnp.transpose(tables, (0, 2, 1)).reshape(EO, V)
    idxT = jnp.transpose(sparse_inputs)
    dT = jnp.transpose(dense_inputs)

    embT = _sc_gather_t(tviewT, idxT)

    w1et = W1[:EO].T.astype(jnp.bfloat16)
    w1dt = W1[EO:].T.astype(jnp.bfloat16)
    w2t = W2.T.astype(jnp.bfloat16)
    w3t = W3.T.astype(jnp.bfloat16)
    bias = (b_lin + b4).reshape(1, 1)
    outT = _mlp_tc_t(
        embT, dT, W_lin, w1et, w1dt, b1.reshape(1024, 1),
        w2t, b2.reshape(512, 1), w3t, b3.reshape(256, 1),
        W4, bias,
    )
    return outT.reshape(B, 1)


# speedup vs baseline: 33.3264x; 1.0014x over previous
import functools

import jax
import jax.numpy as jnp
from jax import lax
from jax.experimental import pallas as pl
from jax.experimental.pallas import tpu as pltpu
from jax.experimental.pallas import tpu_sc as plsc

B = 16384
F = 26
V = 100000
D = 16
EO = F * D
F_DENSE = 13

_NW = 32
_RPW = EO // _NW
_OUT_CHUNK = 4096


def _sc_gather_t(tviewT, idxT):
    mesh = plsc.VectorSubcoreMesh(core_axis_name="core", subcore_axis_name="subcore")

    nchunk = B // _OUT_CHUNK

    @functools.partial(
        pl.kernel,
        out_type=jax.ShapeDtypeStruct((EO, B), jnp.float32),
        mesh=mesh,
        scratch_types=[
            pltpu.VMEM((V,), jnp.float32),
            pltpu.VMEM((128,), jnp.float32),
            pltpu.VMEM((B,), jnp.int32),
            pltpu.VMEM((_OUT_CHUNK,), jnp.float32),
            pltpu.VMEM((_OUT_CHUNK,), jnp.float32),
            pltpu.SemaphoreType.DMA,
            pltpu.SemaphoreType.DMA,
            pltpu.SemaphoreType.DMA,
            pltpu.SemaphoreType.DMA,
        ],
        compiler_params=pltpu.CompilerParams(needs_layout_passes=False),
    )
    def gather_kernel(t_hbm, ttail_hbm, i_hbm, o_hbm, row_v, tail_v, idx_v,
                      out0_v, out1_v, sem_row, sem_idx, sem_o0, sem_o1):
        cid = lax.axis_index("core")
        sid = lax.axis_index("subcore")
        wid = sid * 2 + cid
        outs = (out0_v, out1_v)
        osems = (sem_o0, sem_o1)

        @pl.loop(0, _RPW)
        def _(k):
            r = wid * _RPW + k
            f = r // D
            bounds = (0, 25600, 51200, 76800, 99968)
            rcs = [pltpu.async_copy(ttail_hbm.at[r], tail_v, sem_row)]
            for q in range(4):
                off, ln = bounds[q], bounds[q + 1] - bounds[q]
                rcs.append(pltpu.async_copy(
                    t_hbm.at[r, pl.ds(off, ln)],
                    row_v.at[pl.ds(off, ln)], sem_row))
            ic = pltpu.async_copy(i_hbm.at[f], idx_v, sem_idx)
            ic.wait()
            for rc in rcs:
                rc.wait()
            row_v[pl.ds(99968, 16)] = tail_v[pl.ds(96, 16)]
            row_v[pl.ds(99984, 16)] = tail_v[pl.ds(112, 16)]

            for c in range(nchunk):
                ob = outs[c % 2]
                osem = osems[c % 2]
                if c >= 2:
                    pltpu.make_async_copy(
                        ob, o_hbm.at[r, pl.ds((c - 2) * _OUT_CHUNK, _OUT_CHUNK)], osem
                    ).wait()
                else:
                    @pl.when(k > 0)
                    def _():
                        pltpu.make_async_copy(
                            ob, o_hbm.at[r, pl.ds(c * _OUT_CHUNK, _OUT_CHUNK)], osem
                        ).wait()

                @pl.loop(0, _OUT_CHUNK // 16, unroll=8)
                def _(j):
                    vidx = idx_v[pl.ds(c * _OUT_CHUNK + j * 16, 16)]
                    ob[pl.ds(j * 16, 16)] = plsc.load_gather(row_v, [vidx])

                pltpu.async_copy(
                    ob, o_hbm.at[r, pl.ds(c * _OUT_CHUNK, _OUT_CHUNK)], osem)

        last_r = wid * _RPW + _RPW - 1
        for c in (nchunk - 2, nchunk - 1):
            pltpu.make_async_copy(
                outs[c % 2],
                o_hbm.at[last_r, pl.ds(c * _OUT_CHUNK, _OUT_CHUNK)],
                osems[c % 2],
            ).wait()

    return gather_kernel(tviewT, tviewT[:, V - 128:], idxT)


def _mlp_body_t(g_ref, d_ref, wlin_ref, w1et_ref, w1dt_ref, b1_ref, w2t_ref,
                b2_ref, w3t_ref, b3_ref, w4_ref, bias_ref, o_ref):
    g32 = g_ref[...]
    gb = g32.astype(jnp.bfloat16)
    db = d_ref[...].astype(jnp.bfloat16)
    wide = jnp.sum(g32 * wlin_ref[...], axis=0, keepdims=True)
    h = jnp.dot(w1et_ref[...], gb, preferred_element_type=jnp.float32)
    h = h + jnp.dot(w1dt_ref[...], db, preferred_element_type=jnp.float32)
    h = jnp.maximum(h + b1_ref[...], 0.0)
    h = jnp.dot(w2t_ref[...], h.astype(jnp.bfloat16), preferred_element_type=jnp.float32)
    h = jnp.maximum(h + b2_ref[...], 0.0)
    h = jnp.dot(w3t_ref[...], h.astype(jnp.bfloat16), preferred_element_type=jnp.float32)
    h = jnp.maximum(h + b3_ref[...], 0.0)
    deep = jnp.sum(h * w4_ref[...], axis=0, keepdims=True)
    o_ref[...] = jax.nn.sigmoid(0.5 * (wide + deep + bias_ref[...]))


def _mlp_tc_t(embT, dT, wlin, w1et, w1dt, b1c, w2t, b2c, w3t, b3c, w4, bias):
    bt = 2048
    grid = (B // bt,)

    def _const(shape):
        nd = len(shape)
        return pl.BlockSpec(shape, lambda i, _nd=nd: (0,) * _nd)

    return pl.pallas_call(
        _mlp_body_t,
        grid=grid,
        in_specs=[
            pl.BlockSpec((EO, bt), lambda i: (0, i)),
            pl.BlockSpec((F_DENSE, bt), lambda i: (0, i)),
            _const((EO, 1)),
            _const((1024, EO)),
            _const((1024, F_DENSE)),
            _const((1024, 1)),
            _const((512, 1024)),
            _const((512, 1)),
            _const((256, 512)),
            _const((256, 1)),
            _const((256, 1)),
            _const((1, 1)),
        ],
        out_specs=pl.BlockSpec((1, bt), lambda i: (0, i)),
        out_shape=jax.ShapeDtypeStruct((1, B), jnp.float32),
        compiler_params=pltpu.CompilerParams(
            dimension_semantics=("arbitrary",),
        ),
    )(embT, dT, wlin, w1et, w1dt, b1c, w2t, b2c, w3t, b3c, w4, bias)


def kernel(dense_inputs, sparse_inputs, tables, W_lin, b_lin, W1, b1, W2, b2, W3, b3, W4, b4):
    tviewT = jnp.transpose(tables, (0, 2, 1)).reshape(EO, V)
    idxT = jnp.transpose(sparse_inputs)
    dT = jnp.transpose(dense_inputs)

    embT = _sc_gather_t(tviewT, idxT)

    w1et = W1[:EO].T.astype(jnp.bfloat16)
    w1dt = W1[EO:].T.astype(jnp.bfloat16)
    w2t = W2.T.astype(jnp.bfloat16)
    w3t = W3.T.astype(jnp.bfloat16)
    bias = (b_lin + b4).reshape(1, 1)
    outT = _mlp_tc_t(
        embT, dT, W_lin, w1et, w1dt, b1.reshape(1024, 1),
        w2t, b2.reshape(512, 1), w3t, b3.reshape(256, 1),
        W4, bias,
    )
    return outT.reshape(B, 1)


# speedup vs baseline: 34.7749x; 1.0435x over previous
import functools

import jax
import jax.numpy as jnp
from jax import lax
from jax.experimental import pallas as pl
from jax.experimental.pallas import tpu as pltpu
from jax.experimental.pallas import tpu_sc as plsc

B = 16384
F = 26
V = 100000
D = 16
EO = F * D
F_DENSE = 13

_NW = 32
_RPW = EO // _NW
_OUT_CHUNK = 4096


def _sc_gather_t(tviewT, idxT):
    mesh = plsc.VectorSubcoreMesh(core_axis_name="core", subcore_axis_name="subcore")

    nchunk = B // _OUT_CHUNK

    @functools.partial(
        pl.kernel,
        out_type=jax.ShapeDtypeStruct((EO, B), jnp.float32),
        mesh=mesh,
        scratch_types=[
            pltpu.VMEM((V,), jnp.float32),
            pltpu.VMEM((128,), jnp.float32),
            pltpu.VMEM((B,), jnp.int32),
            pltpu.VMEM((_OUT_CHUNK,), jnp.float32),
            pltpu.VMEM((_OUT_CHUNK,), jnp.float32),
            pltpu.SemaphoreType.DMA,
            pltpu.SemaphoreType.DMA,
            pltpu.SemaphoreType.DMA,
            pltpu.SemaphoreType.DMA,
        ],
        compiler_params=pltpu.CompilerParams(needs_layout_passes=False),
    )
    def gather_kernel(t_hbm, ttail_hbm, i_hbm, o_hbm, row_v, tail_v, idx_v,
                      out0_v, out1_v, sem_row, sem_idx, sem_o0, sem_o1):
        cid = lax.axis_index("core")
        sid = lax.axis_index("subcore")
        wid = sid * 2 + cid
        outs = (out0_v, out1_v)
        osems = (sem_o0, sem_o1)

        @pl.loop(0, _RPW)
        def _(k):
            r = wid * _RPW + k
            f = r // D
            bounds = (0, 25600, 51200, 76800, 99968)
            rcs = [pltpu.async_copy(ttail_hbm.at[r], tail_v, sem_row)]
            for q in range(4):
                off, ln = bounds[q], bounds[q + 1] - bounds[q]
                rcs.append(pltpu.async_copy(
                    t_hbm.at[r, pl.ds(off, ln)],
                    row_v.at[pl.ds(off, ln)], sem_row))
            @pl.when(jnp.logical_or(k == 0, r % D == 0))
            def _():
                pltpu.async_copy(i_hbm.at[f], idx_v, sem_idx).wait()

            for rc in rcs:
                rc.wait()
            row_v[pl.ds(99968, 16)] = tail_v[pl.ds(96, 16)]
            row_v[pl.ds(99984, 16)] = tail_v[pl.ds(112, 16)]

            for c in range(nchunk):
                ob = outs[c % 2]
                osem = osems[c % 2]
                if c >= 2:
                    pltpu.make_async_copy(
                        ob, o_hbm.at[r, pl.ds((c - 2) * _OUT_CHUNK, _OUT_CHUNK)], osem
                    ).wait()
                else:
                    @pl.when(k > 0)
                    def _():
                        pltpu.make_async_copy(
                            ob, o_hbm.at[r, pl.ds(c * _OUT_CHUNK, _OUT_CHUNK)], osem
                        ).wait()

                @pl.loop(0, _OUT_CHUNK // 16, unroll=8)
                def _(j):
                    vidx = idx_v[pl.ds(c * _OUT_CHUNK + j * 16, 16)]
                    ob[pl.ds(j * 16, 16)] = plsc.load_gather(row_v, [vidx])

                pltpu.async_copy(
                    ob, o_hbm.at[r, pl.ds(c * _OUT_CHUNK, _OUT_CHUNK)], osem)

        last_r = wid * _RPW + _RPW - 1
        for c in (nchunk - 2, nchunk - 1):
            pltpu.make_async_copy(
                outs[c % 2],
                o_hbm.at[last_r, pl.ds(c * _OUT_CHUNK, _OUT_CHUNK)],
                osems[c % 2],
            ).wait()

    return gather_kernel(tviewT, tviewT[:, V - 128:], idxT)


def _mlp_body_t(g_ref, d_ref, wlin_ref, w1et_ref, w1dt_ref, b1_ref, w2t_ref,
                b2_ref, w3t_ref, b3_ref, w4_ref, bias_ref, o_ref):
    g32 = g_ref[...]
    gb = g32.astype(jnp.bfloat16)
    db = d_ref[...].astype(jnp.bfloat16)
    wide = jnp.sum(g32 * wlin_ref[...], axis=0, keepdims=True)
    h = jnp.dot(w1et_ref[...], gb, preferred_element_type=jnp.float32)
    h = h + jnp.dot(w1dt_ref[...], db, preferred_element_type=jnp.float32)
    h = jnp.maximum(h + b1_ref[...], 0.0)
    h = jnp.dot(w2t_ref[...], h.astype(jnp.bfloat16), preferred_element_type=jnp.float32)
    h = jnp.maximum(h + b2_ref[...], 0.0)
    h = jnp.dot(w3t_ref[...], h.astype(jnp.bfloat16), preferred_element_type=jnp.float32)
    h = jnp.maximum(h + b3_ref[...], 0.0)
    deep = jnp.sum(h * w4_ref[...], axis=0, keepdims=True)
    o_ref[...] = jax.nn.sigmoid(0.5 * (wide + deep + bias_ref[...]))


def _mlp_tc_t(embT, dT, wlin, w1et, w1dt, b1c, w2t, b2c, w3t, b3c, w4, bias):
    bt = 2048
    grid = (B // bt,)

    def _const(shape):
        nd = len(shape)
        return pl.BlockSpec(shape, lambda i, _nd=nd: (0,) * _nd)

    return pl.pallas_call(
        _mlp_body_t,
        grid=grid,
        in_specs=[
            pl.BlockSpec((EO, bt), lambda i: (0, i)),
            pl.BlockSpec((F_DENSE, bt), lambda i: (0, i)),
            _const((EO, 1)),
            _const((1024, EO)),
            _const((1024, F_DENSE)),
            _const((1024, 1)),
            _const((512, 1024)),
            _const((512, 1)),
            _const((256, 512)),
            _const((256, 1)),
            _const((256, 1)),
            _const((1, 1)),
        ],
        out_specs=pl.BlockSpec((1, bt), lambda i: (0, i)),
        out_shape=jax.ShapeDtypeStruct((1, B), jnp.float32),
        compiler_params=pltpu.CompilerParams(
            dimension_semantics=("arbitrary",),
        ),
    )(embT, dT, wlin, w1et, w1dt, b1c, w2t, b2c, w3t, b3c, w4, bias)


def kernel(dense_inputs, sparse_inputs, tables, W_lin, b_lin, W1, b1, W2, b2, W3, b3, W4, b4):
    tviewT = jnp.transpose(tables, (0, 2, 1)).reshape(EO, V)
    idxT = jnp.transpose(sparse_inputs)
    dT = jnp.transpose(dense_inputs)

    embT = _sc_gather_t(tviewT, idxT)

    w1et = W1[:EO].T.astype(jnp.bfloat16)
    w1dt = W1[EO:].T.astype(jnp.bfloat16)
    w2t = W2.T.astype(jnp.bfloat16)
    w3t = W3.T.astype(jnp.bfloat16)
    bias = (b_lin + b4).reshape(1, 1)
    outT = _mlp_tc_t(
        embT, dT, W_lin, w1et, w1dt, b1.reshape(1024, 1),
        w2t, b2.reshape(512, 1), w3t, b3.reshape(256, 1),
        W4, bias,
    )
    return outT.reshape(B, 1)
